# baseline (speedup 1.0000x reference)
import functools
import math

import jax
import jax.numpy as jnp
from jax import lax
from jax.experimental import pallas as pl
from jax.experimental.pallas import tpu as pltpu
from jax.experimental.pallas import tpu_sc as plsc

_NF = 128
_NP = 100000
_NCQ = 5000
_NBG = 5000
_B = 1024
_SCALAR = 30.0
_AD = 0.25
_AR = 0.25
_BLK = 5000
_KLUT = _NP // _BLK
_T = _KLUT + 1
_C = _SCALAR / math.log(2.0)


def _expmm(w, xt):
    d = jax.lax.dot_general(
        w.astype(jnp.bfloat16), xt, (((1,), (0,)), ((), ())),
        preferred_element_type=jnp.float32)
    return jnp.exp2(d * _C)


def _body(xt_ref, cqb_ref, lut_ref, cq_ref, roi_ref, g_ref,
          cls_ref, det_ref, oim_ref, zb_ref, zn_ref):
    g = pl.program_id(0)

    @pl.when(g == 0)
    def _init():
        eb = _expmm(cqb_ref[...], xt_ref[...])
        zb_ref[...] = jnp.sum(eb, axis=0, keepdims=True)
        ec = _expmm(cq_ref[...], xt_ref[...])
        zn_ref[...] = jnp.sum(ec, axis=0, keepdims=True)

    @pl.when(g > 0)
    def _lut_step():
        e = _expmm(lut_ref[...], xt_ref[...])
        zn_ref[...] += jnp.sum(e, axis=0, keepdims=True)

    @pl.when(g == _T - 1)
    def _finalize():
        zb = zb_ref[...]
        zn = zn_ref[...]
        tot = zb + zn
        c0 = zb / tot
        c1 = zn / tot
        roi = roi_ref[...]
        c_det = jnp.where(roi == -2, c0, c1)
        f_det = _AD * (1.0 - c_det) ** 2.0 * jnp.log(c_det)
        det_ref[0, 0, 0] = -jnp.sum(f_det) / float(_B)
        cls_ref[...] = jnp.concatenate([c0, c1], axis=0)
        ii = jax.lax.broadcasted_iota(jnp.int32, (_NF, _NF), 0)
        jj = jax.lax.broadcasted_iota(jnp.int32, (_NF, _NF), 1)
        diag = ii == jj
        pieces = []
        for c in range(_B // _NF):
            hc = jax.lax.dot_general(
                g_ref[pl.ds(c * _NF, _NF), :].astype(jnp.bfloat16),
                xt_ref[:, pl.ds(c * _NF, _NF)], (((1,), (0,)), ((), ())),
                preferred_element_type=jnp.float32)
            pieces.append(
                jnp.sum(jnp.where(diag, hc, 0.0), axis=0, keepdims=True))
        dlab = jnp.concatenate(pieces, axis=1)
        p = jnp.exp2(dlab * _C) / zn
        per = -_AR * (1.0 - p) ** 2.0 * jnp.log(p)
        validf = (roi >= 0).astype(jnp.float32)
        maskf = (roi >= -1).astype(jnp.float32)
        n_valid = jnp.maximum(jnp.sum(maskf), 1.0)
        oim_vec = per * validf * c1 * c1
        oim_ref[0, 0, 0] = jnp.sum(oim_vec) / n_valid


@functools.partial(jax.jit, static_argnames=())
def _run(xt, cqb, lut, cq, roi, gat):
    return pl.pallas_call(
        _body,
        grid=(_T,),
        in_specs=[
            pl.BlockSpec((_NF, _B), lambda g: (0, 0)),
            pl.BlockSpec((_NBG, _NF), lambda g: (0, 0)),
            pl.BlockSpec((_BLK, _NF), lambda g: (jnp.clip(g - 1, 0, _KLUT - 1), 0)),
            pl.BlockSpec((_NCQ, _NF), lambda g: (0, 0)),
            pl.BlockSpec((1, _B), lambda g: (0, 0)),
            pl.BlockSpec((_B, _NF), lambda g: (0, 0)),
        ],
        out_specs=[
            pl.BlockSpec((2, _B), lambda g: (0, 0)),
            pl.BlockSpec((1, 1, 1), lambda g: (0, 0, 0), memory_space=pltpu.SMEM),
            pl.BlockSpec((1, 1, 1), lambda g: (0, 0, 0), memory_space=pltpu.SMEM),
        ],
        out_shape=[
            jax.ShapeDtypeStruct((2, _B), jnp.float32),
            jax.ShapeDtypeStruct((1, 1, 1), jnp.float32),
            jax.ShapeDtypeStruct((1, 1, 1), jnp.float32),
        ],
        scratch_shapes=[
            pltpu.VMEM((1, _B), jnp.float32),
            pltpu.VMEM((1, _B), jnp.float32),
        ],
    )(xt, cqb, lut, cq, roi, gat)


@functools.cache
def _sc_gather():
    info = plsc.get_sparse_core_info()
    ncores = info.num_cores
    nw = ncores * info.num_subcores
    b_per_w = _B // nw
    mesh = plsc.VectorSubcoreMesh(core_axis_name="c", subcore_axis_name="s")

    @functools.partial(
        pl.kernel,
        out_type=jax.ShapeDtypeStruct((_B, _NF), jnp.float32),
        mesh=mesh,
        scratch_types=[
            pltpu.VMEM((b_per_w,), jnp.int32),
            pltpu.VMEM((b_per_w, _NF), jnp.float32),
            pltpu.SemaphoreType.DMA,
        ],
    )
    def gather(lut_hbm, lab_hbm, out_hbm, idx_v, rows_v, sem):
        wid = lax.axis_index("s") * ncores + lax.axis_index("c")
        base = wid * b_per_w
        pltpu.sync_copy(lab_hbm.at[pl.ds(base, b_per_w)], idx_v)
        pltpu.async_copy(lut_hbm.at[idx_v], rows_v, sem).wait()
        pltpu.sync_copy(rows_v, out_hbm.at[pl.ds(base, b_per_w)])

    return gather


def kernel(inputs, roi_label, lut, cq, cqb):
    xt = inputs.T.astype(jnp.bfloat16)
    roi = roi_label.astype(jnp.int32).reshape(1, _B)
    lab1 = jnp.clip(roi_label.astype(jnp.int32), 0, _NP - 1)
    gat = _sc_gather()(lut, lab1)
    cls_t, det, oim = _run(xt, cqb, lut, cq, roi, gat)
    return cls_t.T, det.reshape(()), oim.reshape(())


# speedup vs baseline: 1.0111x; 1.0111x over previous
import functools
import math

import jax
import jax.numpy as jnp
from jax import lax
from jax.experimental import pallas as pl
from jax.experimental.pallas import tpu as pltpu
from jax.experimental.pallas import tpu_sc as plsc

_NF = 128
_NP = 100000
_NCQ = 5000
_NBG = 5000
_B = 1024
_SCALAR = 30.0
_AD = 0.25
_AR = 0.25
_BLK = 5000
_KLUT = _NP // _BLK
_T = _KLUT + 1
_C = _SCALAR / math.log(2.0)


def _expmm(w, xt):
    d = jax.lax.dot_general(
        w.astype(jnp.bfloat16), xt, (((1,), (0,)), ((), ())),
        preferred_element_type=jnp.float32)
    return jnp.exp2(d * _C)


def _body(xt_ref, cqb_ref, lut_ref, cq_ref, roi_ref,
          cls_ref, det_ref, zbo_ref, zno_ref, zb_ref, zn_ref):
    g = pl.program_id(0)

    @pl.when(g == 0)
    def _init():
        eb = _expmm(cqb_ref[...], xt_ref[...])
        zb_ref[...] = jnp.sum(eb, axis=0, keepdims=True)
        ec = _expmm(cq_ref[...], xt_ref[...])
        zn_ref[...] = jnp.sum(ec, axis=0, keepdims=True)

    @pl.when(g > 0)
    def _lut_step():
        e = _expmm(lut_ref[...], xt_ref[...])
        zn_ref[...] += jnp.sum(e, axis=0, keepdims=True)

    @pl.when(g == _T - 1)
    def _finalize():
        zb = zb_ref[...]
        zn = zn_ref[...]
        tot = zb + zn
        c0 = zb / tot
        c1 = zn / tot
        roi = roi_ref[...]
        c_det = jnp.where(roi == -2, c0, c1)
        f_det = _AD * (1.0 - c_det) ** 2.0 * jnp.log(c_det)
        det_ref[0, 0, 0] = -jnp.sum(f_det) / float(_B)
        zbo_ref[...] = zb
        zno_ref[...] = zn
        cls_ref[...] = jnp.concatenate([c0, c1], axis=0)


@functools.partial(jax.jit, static_argnames=())
def _run(xt, cqb, lut, cq, roi):
    return pl.pallas_call(
        _body,
        grid=(_T,),
        in_specs=[
            pl.BlockSpec((_NF, _B), lambda g: (0, 0)),
            pl.BlockSpec((_NBG, _NF), lambda g: (0, 0)),
            pl.BlockSpec((_BLK, _NF), lambda g: (jnp.clip(g - 1, 0, _KLUT - 1), 0)),
            pl.BlockSpec((_NCQ, _NF), lambda g: (0, 0)),
            pl.BlockSpec((1, _B), lambda g: (0, 0)),
        ],
        out_specs=[
            pl.BlockSpec((2, _B), lambda g: (0, 0)),
            pl.BlockSpec((1, 1, 1), lambda g: (0, 0, 0), memory_space=pltpu.SMEM),
            pl.BlockSpec((1, _B), lambda g: (0, 0)),
            pl.BlockSpec((1, _B), lambda g: (0, 0)),
        ],
        out_shape=[
            jax.ShapeDtypeStruct((2, _B), jnp.float32),
            jax.ShapeDtypeStruct((1, 1, 1), jnp.float32),
            jax.ShapeDtypeStruct((1, _B), jnp.float32),
            jax.ShapeDtypeStruct((1, _B), jnp.float32),
        ],
        scratch_shapes=[
            pltpu.VMEM((1, _B), jnp.float32),
            pltpu.VMEM((1, _B), jnp.float32),
        ],
    )(xt, cqb, lut, cq, roi)


@functools.cache
def _sc_gather():
    info = plsc.get_sparse_core_info()
    ncores = info.num_cores
    nw = ncores * info.num_subcores
    b_per_w = _B // nw
    mesh = plsc.VectorSubcoreMesh(core_axis_name="c", subcore_axis_name="s")

    @functools.partial(
        pl.kernel,
        out_type=jax.ShapeDtypeStruct((_B, _NF), jnp.float32),
        mesh=mesh,
        scratch_types=[
            pltpu.VMEM((b_per_w,), jnp.int32),
            pltpu.VMEM((b_per_w, _NF), jnp.float32),
            pltpu.SemaphoreType.DMA,
        ],
    )
    def gather(lut_hbm, lab_hbm, out_hbm, idx_v, rows_v, sem):
        wid = lax.axis_index("s") * ncores + lax.axis_index("c")
        base = wid * b_per_w
        pltpu.sync_copy(lab_hbm.at[pl.ds(base, b_per_w)], idx_v)
        pltpu.async_copy(lut_hbm.at[idx_v], rows_v, sem).wait()
        pltpu.sync_copy(rows_v, out_hbm.at[pl.ds(base, b_per_w)])

    return gather


def _oim_body(g_ref, xt_ref, zb_ref, zn_ref, roi_ref, oim_ref):
    ii = jax.lax.broadcasted_iota(jnp.int32, (_NF, _NF), 0)
    jj = jax.lax.broadcasted_iota(jnp.int32, (_NF, _NF), 1)
    diag = ii == jj
    pieces = []
    for c in range(_B // _NF):
        hc = jax.lax.dot_general(
            g_ref[pl.ds(c * _NF, _NF), :].astype(jnp.bfloat16),
            xt_ref[:, pl.ds(c * _NF, _NF)], (((1,), (0,)), ((), ())),
            preferred_element_type=jnp.float32)
        pieces.append(jnp.sum(jnp.where(diag, hc, 0.0), axis=0, keepdims=True))
    dlab = jnp.concatenate(pieces, axis=1)
    zb = zb_ref[...]
    zn = zn_ref[...]
    c1 = zn / (zb + zn)
    roi = roi_ref[...]
    p = jnp.exp2(dlab * _C) / zn
    per = -_AR * (1.0 - p) ** 2.0 * jnp.log(p)
    validf = (roi >= 0).astype(jnp.float32)
    maskf = (roi >= -1).astype(jnp.float32)
    n_valid = jnp.maximum(jnp.sum(maskf), 1.0)
    oim_vec = per * validf * c1 * c1
    oim_ref[0, 0, 0] = jnp.sum(oim_vec) / n_valid


@functools.partial(jax.jit, static_argnames=())
def _oim_run(gat, xt, zb, zn, roi):
    return pl.pallas_call(
        _oim_body,
        grid=(1,),
        in_specs=[
            pl.BlockSpec((_B, _NF), lambda g: (0, 0)),
            pl.BlockSpec((_NF, _B), lambda g: (0, 0)),
            pl.BlockSpec((1, _B), lambda g: (0, 0)),
            pl.BlockSpec((1, _B), lambda g: (0, 0)),
            pl.BlockSpec((1, _B), lambda g: (0, 0)),
        ],
        out_specs=pl.BlockSpec((1, 1, 1), lambda g: (0, 0, 0),
                               memory_space=pltpu.SMEM),
        out_shape=jax.ShapeDtypeStruct((1, 1, 1), jnp.float32),
    )(gat, xt, zb, zn, roi)


def kernel(inputs, roi_label, lut, cq, cqb):
    xt = inputs.T.astype(jnp.bfloat16)
    roi = roi_label.astype(jnp.int32).reshape(1, _B)
    lab1 = jnp.clip(roi_label.astype(jnp.int32), 0, _NP - 1)
    gat = _sc_gather()(lut, lab1)
    cls_t, det, zb, zn = _run(xt, cqb, lut, cq, roi)
    oim = _oim_run(gat, xt, zb, zn, roi)
    return cls_t.T, det.reshape(()), oim.reshape(())
